# baseline (speedup 1.0000x reference)
import functools

import jax
import jax.numpy as jnp
import numpy as np
from jax import lax
from jax.experimental import pallas as pl
from jax.experimental.pallas import tpu as pltpu
from jax.experimental.pallas import tpu_sc as plsc

N = 10000
E = 320000
M = 128
H = 128

NC = 2
NS = 16
HC = M // NC
WPR = HC // 2
EPT = E // NS
B = 80
NB = EPT // B
LANES = 16
G16 = B // LANES
HCHUNKS = HC // LANES
ROWS_PER_TILE = 632

NBUF = 4
NITER = 62

_half = np.zeros(HC, np.int32)
for _k in range(2):
    for _i in range(LANES):
        _half[32 * _k + _i] = 32 * _k + 2 * _i
        _half[32 * _k + LANES + _i] = 32 * _k + 2 * _i + 1
_W_PERM = np.concatenate([_half, HC + _half])

def _sc_aggregate_body(src_hbm, dst_hbm, w_hbm, xi_hbm, part_hbm,
                       src_v, dst_v, w_v, gbufs, sbufs, acc, gsems, ssems):
    c = lax.axis_index("c")
    s = lax.axis_index("s")

    pltpu.sync_copy(src_hbm.at[s], src_v)
    pltpu.sync_copy(dst_hbm.at[s], dst_v)
    pltpu.sync_copy(w_hbm.at[s], w_v)

    def _fix(j, _):
        for g in range(G16):
            sl = pl.ds(g * LANES, LANES)
            src_v[j, sl] = src_v[j, sl] * 2 + c
        return 0

    lax.fori_loop(0, NB, _fix, 0)

    zeros16 = jnp.zeros((LANES,), jnp.float32)
    s0 = sbufs[0]

    def _zrow(i, _):
        for h in range(HCHUNKS):
            s0[i, pl.ds(h * LANES, LANES)] = zeros16
        return 0

    lax.fori_loop(0, B, _zrow, 0)
    base_row = pl.multiple_of(
        jnp.minimum(s * ROWS_PER_TILE, N - ROWS_PER_TILE), 8)
    full, rem = divmod(ROWS_PER_TILE, B)
    for k in range(full):
        pltpu.sync_copy(s0, acc.at[pl.ds(base_row + k * B, B)])
    if rem:
        pltpu.sync_copy(s0.at[pl.ds(0, rem)],
                        acc.at[pl.ds(base_row + full * B, rem)])
    plsc.subcore_barrier()

    himask = jnp.full((LANES,), -65536, jnp.int32)

    def _scale(gb, sb, j):
        for g in range(G16):
            wvec = w_v[j, pl.ds(g * LANES, LANES)]
            for r in range(LANES):
                wr = wvec[r]
                row = g * LANES + r
                for k in range(2):
                    v = gb[row, pl.ds(k * LANES, LANES)]
                    a = plsc.bitcast(lax.shift_left(v, 16), jnp.float32)
                    b = plsc.bitcast(lax.bitwise_and(v, himask), jnp.float32)
                    sb[row, pl.ds(32 * k, LANES)] = a * wr
                    sb[row, pl.ds(32 * k + LANES, LANES)] = b * wr

    def _gather(p, j):
        pltpu.async_copy(xi_hbm.at[src_v.at[j]], gbufs[p], gsems[p])

    def _gather_wait(p, j):
        pltpu.make_async_copy(xi_hbm.at[src_v.at[j]], gbufs[p], gsems[p]).wait()

    def _scatter(p, j):
        pltpu.async_copy(sbufs[p % 2], acc.at[dst_v.at[j]], ssems[p % 2],
                         add=True)

    def _scatter_wait(p, j):
        pltpu.make_async_copy(sbufs[p % 2], acc.at[dst_v.at[j]],
                              ssems[p % 2]).wait()

    _gather(0, 0)
    _gather(1, 1)
    _gather(2, 2)

    def _step(jj, _):
        j0 = jj * NBUF
        for p in range(NBUF):
            j = j0 + p

            @pl.when(j + 3 < NB)
            def _():
                _gather((p + 3) % NBUF, j + 3)

            _gather_wait(p, j)

            @pl.when(j >= 2)
            def _():
                _scatter_wait(p, j - 2)

            _scale(gbufs[p], sbufs[p % 2], j)
            _scatter(p, j)
        return 0

    lax.fori_loop(0, NITER, _step, 0)

    jE = NBUF * NITER
    _gather_wait(0, jE)
    _scatter_wait(0, jE - 2)
    _scale(gbufs[0], sbufs[0], jE)
    _scatter(0, jE)
    _gather_wait(1, jE + 1)
    _scatter_wait(1, jE - 1)
    _scale(gbufs[1], sbufs[1], jE + 1)
    _scatter(1, jE + 1)
    _scatter_wait(0, jE)
    _scatter_wait(1, jE + 1)

    plsc.subcore_barrier()
    rows = pl.ds(base_row, ROWS_PER_TILE)
    pltpu.sync_copy(acc.at[rows], part_hbm.at[c].at[rows])


_sc_aggregate = functools.partial(
    pl.kernel,
    out_type=jax.ShapeDtypeStruct((NC, N, HC), jnp.float32),
    mesh=plsc.VectorSubcoreMesh(core_axis_name="c", subcore_axis_name="s"),
    compiler_params=pltpu.CompilerParams(use_tc_tiling_on_sc=False,
                                         needs_layout_passes=False),
    scratch_types=[
        pltpu.VMEM((NB, B), jnp.int32),
        pltpu.VMEM((NB, B), jnp.int32),
        pltpu.VMEM((NB, B), jnp.float32),
        [pltpu.VMEM((B, WPR), jnp.int32)] * NBUF,
        [pltpu.VMEM((B, HC), jnp.float32)] * 2,
        pltpu.VMEM_SHARED((N, HC), jnp.float32),
        [pltpu.SemaphoreType.DMA] * NBUF,
        [pltpu.SemaphoreType.DMA] * 2,
    ],
)(_sc_aggregate_body)


BM = 2000


def _tc_combine_body(p_ref, w_ref, o_ref):
    acc = (jnp.dot(p_ref[0], w_ref[:HC],
                   preferred_element_type=jnp.float32) +
           jnp.dot(p_ref[1], w_ref[HC:],
                   preferred_element_type=jnp.float32))
    o_ref[...] = jnp.maximum(acc, 0.0)


def _tc_combine(partials, W2):
    return pl.pallas_call(
        _tc_combine_body,
        grid=(N // BM,),
        in_specs=[
            pl.BlockSpec((NC, BM, HC), lambda i: (0, i, 0)),
            pl.BlockSpec((M, H), lambda i: (0, 0)),
        ],
        out_specs=pl.BlockSpec((BM, H), lambda i: (i, 0)),
        out_shape=jax.ShapeDtypeStruct((N, H), jnp.float32),
    )(partials, W2)


@jax.jit
def kernel(edge_index, edge_weight, X, W):
    dst = edge_index[0].reshape(NS, NB, B)
    src = edge_index[1].reshape(NS, NB, B)
    w = edge_weight.reshape(NS, NB, B)
    xb = X.astype(jnp.bfloat16).reshape(N, HC, 2)
    xi = lax.bitcast_convert_type(xb, jnp.int32).reshape(NC * N, WPR)
    w2 = W[_W_PERM, :]
    partials = _sc_aggregate(src, dst, w, xi)
    return _tc_combine(partials, w2)


# speedup vs baseline: 1.1108x; 1.1108x over previous
import functools

import jax
import jax.numpy as jnp
import numpy as np
from jax import lax
from jax.experimental import pallas as pl
from jax.experimental.pallas import tpu as pltpu
from jax.experimental.pallas import tpu_sc as plsc

N = 10000
E = 320000
M = 128
H = 128

NC = 2
NS = 16
HC = M // NC
WPR = HC // 2
EPT = E // NS
B = 80
NB = EPT // B
LANES = 16
G16 = B // LANES
HCHUNKS = HC // LANES
ROWS_PER_TILE = 632

NBUF = 4
NITER = 62

def _sc_aggregate_body(src_hbm, dst_hbm, w_hbm, xi_hbm, part_hbm,
                       src_v, dst_v, w_v, gbufs, sbufs, acc, gsems, ssems):
    c = lax.axis_index("c")
    s = lax.axis_index("s")

    pltpu.sync_copy(src_hbm.at[s], src_v)
    pltpu.sync_copy(dst_hbm.at[s], dst_v)
    pltpu.sync_copy(w_hbm.at[s], w_v)

    def _fix(j, _):
        for g in range(G16):
            sl = pl.ds(g * LANES, LANES)
            src_v[j, sl] = src_v[j, sl] * 2 + c
        return 0

    lax.fori_loop(0, NB, _fix, 0)

    zeros16 = jnp.zeros((LANES,), jnp.float32)
    s0 = sbufs[0]

    def _zrow(i, _):
        for h in range(HCHUNKS):
            s0[i, pl.ds(h * LANES, LANES)] = zeros16
        return 0

    lax.fori_loop(0, B, _zrow, 0)
    base_row = pl.multiple_of(
        jnp.minimum(s * ROWS_PER_TILE, N - ROWS_PER_TILE), 8)
    full, rem = divmod(ROWS_PER_TILE, B)
    for k in range(full):
        pltpu.sync_copy(s0, acc.at[pl.ds(base_row + k * B, B)])
    if rem:
        pltpu.sync_copy(s0.at[pl.ds(0, rem)],
                        acc.at[pl.ds(base_row + full * B, rem)])
    plsc.subcore_barrier()

    def _scale(gb, sb, j):
        for g in range(G16):
            wvec = w_v[j, pl.ds(g * LANES, LANES)]
            for r in range(LANES):
                wr = wvec[r]
                row = g * LANES + r
                for h in range(HCHUNKS):
                    sl = pl.ds(h * LANES, LANES)
                    sb[row, sl] = gb[row, sl] * wr

    HB = B // 2

    def _gather(p, j):
        pltpu.async_copy(xi_hbm.at[src_v.at[j, pl.ds(0, HB)]],
                         gbufs[p].at[pl.ds(0, HB)], gsems[p])
        pltpu.async_copy(xi_hbm.at[src_v.at[j, pl.ds(HB, HB)]],
                         gbufs[p].at[pl.ds(HB, HB)], gsems[p])

    def _gather_wait(p, j):
        pltpu.make_async_copy(xi_hbm.at[src_v.at[j, pl.ds(0, HB)]],
                              gbufs[p].at[pl.ds(0, HB)], gsems[p]).wait()
        pltpu.make_async_copy(xi_hbm.at[src_v.at[j, pl.ds(HB, HB)]],
                              gbufs[p].at[pl.ds(HB, HB)], gsems[p]).wait()

    def _scatter(p, j):
        pltpu.async_copy(sbufs[p % 2], acc.at[dst_v.at[j]], ssems[p % 2],
                         add=True)

    def _scatter_wait(p, j):
        pltpu.make_async_copy(sbufs[p % 2], acc.at[dst_v.at[j]],
                              ssems[p % 2]).wait()

    _gather(0, 0)
    _gather(1, 1)
    _gather(2, 2)

    def _step(jj, _):
        j0 = jj * NBUF
        for p in range(NBUF):
            j = j0 + p

            @pl.when(j + 3 < NB)
            def _():
                _gather((p + 3) % NBUF, j + 3)

            _gather_wait(p, j)

            @pl.when(j >= 2)
            def _():
                _scatter_wait(p, j - 2)

            _scale(gbufs[p], sbufs[p % 2], j)
            _scatter(p, j)
        return 0

    lax.fori_loop(0, NITER, _step, 0)

    jE = NBUF * NITER
    _gather_wait(0, jE)
    _scatter_wait(0, jE - 2)
    _scale(gbufs[0], sbufs[0], jE)
    _scatter(0, jE)
    _gather_wait(1, jE + 1)
    _scatter_wait(1, jE - 1)
    _scale(gbufs[1], sbufs[1], jE + 1)
    _scatter(1, jE + 1)
    _scatter_wait(0, jE)
    _scatter_wait(1, jE + 1)

    plsc.subcore_barrier()
    rows = pl.ds(base_row, ROWS_PER_TILE)
    pltpu.sync_copy(acc.at[rows], part_hbm.at[c].at[rows])


_sc_aggregate = functools.partial(
    pl.kernel,
    out_type=jax.ShapeDtypeStruct((NC, N, HC), jnp.float32),
    mesh=plsc.VectorSubcoreMesh(core_axis_name="c", subcore_axis_name="s"),
    compiler_params=pltpu.CompilerParams(use_tc_tiling_on_sc=False),
    scratch_types=[
        pltpu.VMEM((NB, B), jnp.int32),
        pltpu.VMEM((NB, B), jnp.int32),
        pltpu.VMEM((NB, B), jnp.float32),
        [pltpu.VMEM((B, HC), jnp.float32)] * NBUF,
        [pltpu.VMEM((B, HC), jnp.float32)] * 2,
        pltpu.VMEM_SHARED((N, HC), jnp.float32),
        [pltpu.SemaphoreType.DMA] * NBUF,
        [pltpu.SemaphoreType.DMA] * 2,
    ],
)(_sc_aggregate_body)


BM = 2000


def _tc_combine_body(p_ref, w_ref, o_ref):
    acc = (jnp.dot(p_ref[0], w_ref[:HC],
                   preferred_element_type=jnp.float32) +
           jnp.dot(p_ref[1], w_ref[HC:],
                   preferred_element_type=jnp.float32))
    o_ref[...] = jnp.maximum(acc, 0.0)


def _tc_combine(partials, W2):
    return pl.pallas_call(
        _tc_combine_body,
        grid=(N // BM,),
        in_specs=[
            pl.BlockSpec((NC, BM, HC), lambda i: (0, i, 0)),
            pl.BlockSpec((M, H), lambda i: (0, 0)),
        ],
        out_specs=pl.BlockSpec((BM, H), lambda i: (i, 0)),
        out_shape=jax.ShapeDtypeStruct((N, H), jnp.float32),
    )(partials, W2)


@jax.jit
def kernel(edge_index, edge_weight, X, W):
    dst = edge_index[0].reshape(NS, NB, B)
    src = edge_index[1].reshape(NS, NB, B)
    w = edge_weight.reshape(NS, NB, B)
    x2 = X.reshape(NC * N, HC)
    partials = _sc_aggregate(src, dst, w, x2)
    return _tc_combine(partials, W)


# speedup vs baseline: 1.1180x; 1.0064x over previous
import functools

import jax
import jax.numpy as jnp
from jax import lax
from jax.experimental import pallas as pl
from jax.experimental.pallas import tpu as pltpu
from jax.experimental.pallas import tpu_sc as plsc

N = 10000
E = 320000
M = 128
H = 128

NC = 2
NS = 16
HC = M // NC
EPT = E // NS
B = 80
NB = EPT // B
LANES = 16
G16 = B // LANES
HCHUNKS = HC // LANES
ROWS_PER_TILE = 632

NBUF = 4
NITER = 62

def _sc_aggregate_body(src_hbm, dst_hbm, w_hbm, xi_hbm, part_hbm,
                       src_v, dst_v, w_v, gbufs, sbufs, acc, gsems, ssems):
    c = lax.axis_index("c")
    s = lax.axis_index("s")

    pltpu.sync_copy(src_hbm.at[s], src_v)
    pltpu.sync_copy(dst_hbm.at[s], dst_v)
    pltpu.sync_copy(w_hbm.at[s], w_v)

    def _fix(j, _):
        for g in range(G16):
            sl = pl.ds(g * LANES, LANES)
            src_v[j, sl] = src_v[j, sl] * 2 + c
        return 0

    lax.fori_loop(0, NB, _fix, 0)

    zeros16 = jnp.zeros((LANES,), jnp.float32)
    s0 = sbufs[0]

    def _zrow(i, _):
        for h in range(HCHUNKS):
            s0[i, pl.ds(h * LANES, LANES)] = zeros16
        return 0

    lax.fori_loop(0, B, _zrow, 0)
    base_row = pl.multiple_of(
        jnp.minimum(s * ROWS_PER_TILE, N - ROWS_PER_TILE), 8)
    full, rem = divmod(ROWS_PER_TILE, B)
    for k in range(full):
        pltpu.sync_copy(s0, acc.at[pl.ds(base_row + k * B, B)])
    if rem:
        pltpu.sync_copy(s0.at[pl.ds(0, rem)],
                        acc.at[pl.ds(base_row + full * B, rem)])
    plsc.subcore_barrier()

    def _scale(gb, sb, j):
        for g in range(G16):
            wvec = w_v[j, pl.ds(g * LANES, LANES)]
            for r in range(LANES):
                wr = wvec[r]
                row = g * LANES + r
                for h in range(HCHUNKS):
                    sl = pl.ds(h * LANES, LANES)
                    sb[row, sl] = gb[row, sl] * wr

    def _gather(p, j):
        pltpu.async_copy(xi_hbm.at[src_v.at[j]], gbufs[p], gsems[p])

    def _gather_wait(p, j):
        pltpu.make_async_copy(xi_hbm.at[src_v.at[j]], gbufs[p], gsems[p]).wait()

    def _scatter(p, j):
        pltpu.async_copy(sbufs[p % 2], acc.at[dst_v.at[j]], ssems[p % 2],
                         add=True)

    def _scatter_wait(p, j):
        pltpu.make_async_copy(sbufs[p % 2], acc.at[dst_v.at[j]],
                              ssems[p % 2]).wait()

    _gather(0, 0)
    _gather(1, 1)
    _gather(2, 2)

    def _step(jj, _):
        j0 = jj * NBUF
        for p in range(NBUF):
            j = j0 + p

            @pl.when(j + 3 < NB)
            def _():
                _gather((p + 3) % NBUF, j + 3)

            _gather_wait(p, j)

            @pl.when(j >= 2)
            def _():
                _scatter_wait(p, j - 2)

            _scale(gbufs[p], sbufs[p % 2], j)
            _scatter(p, j)
        return 0

    lax.fori_loop(0, NITER, _step, 0)

    jE = NBUF * NITER
    _gather_wait(0, jE)
    _scatter_wait(0, jE - 2)
    _scale(gbufs[0], sbufs[0], jE)
    _scatter(0, jE)
    _gather_wait(1, jE + 1)
    _scatter_wait(1, jE - 1)
    _scale(gbufs[1], sbufs[1], jE + 1)
    _scatter(1, jE + 1)
    _scatter_wait(0, jE)
    _scatter_wait(1, jE + 1)

    plsc.subcore_barrier()
    rows = pl.ds(base_row, ROWS_PER_TILE)
    pltpu.sync_copy(acc.at[rows], part_hbm.at[c].at[rows])


_sc_aggregate = functools.partial(
    pl.kernel,
    out_type=jax.ShapeDtypeStruct((NC, N, HC), jnp.float32),
    mesh=plsc.VectorSubcoreMesh(core_axis_name="c", subcore_axis_name="s"),
    compiler_params=pltpu.CompilerParams(use_tc_tiling_on_sc=False),
    scratch_types=[
        pltpu.VMEM((NB, B), jnp.int32),
        pltpu.VMEM((NB, B), jnp.int32),
        pltpu.VMEM((NB, B), jnp.float32),
        [pltpu.VMEM((B, HC), jnp.float32)] * NBUF,
        [pltpu.VMEM((B, HC), jnp.float32)] * 2,
        pltpu.VMEM_SHARED((N, HC), jnp.float32),
        [pltpu.SemaphoreType.DMA] * NBUF,
        [pltpu.SemaphoreType.DMA] * 2,
    ],
)(_sc_aggregate_body)


BM = 2000


def _tc_combine_body(p_ref, w_ref, o_ref):
    acc = (jnp.dot(p_ref[0], w_ref[:HC],
                   preferred_element_type=jnp.float32) +
           jnp.dot(p_ref[1], w_ref[HC:],
                   preferred_element_type=jnp.float32))
    o_ref[...] = jnp.maximum(acc, 0.0)


def _tc_combine(partials, W2):
    return pl.pallas_call(
        _tc_combine_body,
        grid=(N // BM,),
        in_specs=[
            pl.BlockSpec((NC, BM, HC), lambda i: (0, i, 0)),
            pl.BlockSpec((M, H), lambda i: (0, 0)),
        ],
        out_specs=pl.BlockSpec((BM, H), lambda i: (i, 0)),
        out_shape=jax.ShapeDtypeStruct((N, H), jnp.float32),
    )(partials, W2)


@jax.jit
def kernel(edge_index, edge_weight, X, W):
    dst = edge_index[0].reshape(NS, NB, B)
    src = edge_index[1].reshape(NS, NB, B)
    w = edge_weight.reshape(NS, NB, B)
    x2 = X.reshape(NC * N, HC)
    partials = _sc_aggregate(src, dst, w, x2)
    return _tc_combine(partials, W)
